# baseline (speedup 1.0000x reference)
import functools

import jax
import jax.numpy as jnp
from jax import lax
from jax.experimental import pallas as pl
from jax.experimental.pallas import tpu as pltpu
from jax.experimental.pallas import tpu_sc as plsc

_ROWS = 128
_EMB = 49152
_BATCH = 8
_NUM_WORKERS = 32
_ROWS_PER_WORKER = _ROWS // _NUM_WORKERS

_mesh = plsc.VectorSubcoreMesh(core_axis_name="c", subcore_axis_name="s")


@functools.partial(
    pl.kernel,
    out_type=jax.ShapeDtypeStruct((_BATCH, _ROWS, _EMB), jnp.float32),
    mesh=_mesh,
    scratch_types=[
        pltpu.VMEM((2, _EMB), jnp.float32),
        pltpu.SemaphoreType.DMA,
        pltpu.SemaphoreType.DMA,
    ],
)
def _broadcast_table(table_hbm, out_hbm, buf, in_sem, out_sem):
    wid = lax.axis_index("s") * 2 + lax.axis_index("c")
    base = wid * _ROWS_PER_WORKER

    pending_writes = []
    for chunk in range(_ROWS_PER_WORKER // 2):
        row = base + 2 * chunk
        for w in pending_writes:
            w.wait()
        pltpu.async_copy(table_hbm.at[pl.ds(row, 2)], buf, in_sem).wait()
        pending_writes = [
            pltpu.async_copy(
                buf,
                out_hbm.at[b].at[pl.ds(row, 2)],
                out_sem,
            )
            for b in range(_BATCH)
        ]
    for w in pending_writes:
        w.wait()


def kernel(batch_size, table):
    del batch_size
    return _broadcast_table(table)
